# baseline (speedup 1.0000x reference)
import numpy as np
import jax
import jax.numpy as jnp
from jax import lax
from jax.experimental import pallas as pl
from jax.experimental.pallas import tpu as pltpu
from jax.experimental.pallas import tpu_sc as plsc

N = 10000
E = 320000
D = 128
DE = 16

NC = 2
NS = 16
L = 16
NW = NC * NS
BE = 64
EPW = 10048
EP = EPW * NW
NIT = EPW // BE
NPAD = 10240
RPT = NPAD // NS
RCH = BE
NCH = RPT // RCH

BN = 1024
BNC = 1000
BEB = 5024

_PERM = np.empty((D,), dtype=np.int32)
for _c in range(D // 32):
    for _i in range(16):
        _PERM[32 * _c + 2 * _i] = 32 * _c + _i
        _PERM[32 * _c + 2 * _i + 1] = 32 * _c + 16 + _i


def _proj_body(x_ref, w_ref, b_ref, xk_ref, xq_ref, xv_ref, sk_ref):
    acc = jnp.dot(x_ref[...], w_ref[...], preferred_element_type=jnp.float32)
    xk_ref[...] = acc[:, 0:D]
    xq_ref[...] = acc[:, D:2 * D]
    xv_ref[...] = acc[:, 2 * D:3 * D]
    sk_ref[...] = acc[:, 3 * D:4 * D] + b_ref[...]


def _edge_body(ea_ref, wg_ref, wv_ref, bg_ref, bv_ref, eg_ref, ev_ref):
    ea = ea_ref[...]
    eg = jnp.dot(ea, wg_ref[...], preferred_element_type=jnp.float32) + bg_ref[...]
    ev = jnp.dot(ea, wv_ref[...], preferred_element_type=jnp.float32) + bv_ref[...]
    eg_ref[...] = eg.astype(jnp.bfloat16)
    ev_ref[...] = ev.astype(jnp.bfloat16)


def _comb_body(p_ref, s_ref, o_ref):
    o_ref[...] = p_ref[0] + p_ref[1] + s_ref[...]


def _sc_body(xk_hbm, xq_hbm, xv_hbm, eg_hbm, ev_hbm, src_hbm, dst_hbm, out_hbm,
             didx, sidx, kbuf, qbuf, v2buf, zb, vb, mbuf, shared,
             sem_idx, sem):
    core = lax.axis_index("c")
    sid = lax.axis_index("s")
    wid = core * NS + sid

    def zrow(r, carry):
        for c in range(D // L):
            mbuf[r, pl.ds(c * L, L)] = jnp.zeros((L,), jnp.float32)
        return carry

    lax.fori_loop(0, RCH, zrow, 0)
    r0 = sid * RPT
    for j in range(NCH):
        pltpu.sync_copy(mbuf, shared.at[pl.ds(r0 + j * RCH, RCH)])
    plsc.subcore_barrier()

    ebase = wid * EPW

    def it_body(i, carry):
        e0 = ebase + i * BE
        cpd = pltpu.async_copy(dst_hbm.at[pl.ds(e0, BE)], didx, sem_idx)
        cps = pltpu.async_copy(src_hbm.at[pl.ds(e0, BE)], sidx, sem_idx)
        cpg = pltpu.async_copy(eg_hbm.at[pl.ds(e0 * (D // 2), BE * (D // 2))], zb, sem)
        cpv = pltpu.async_copy(ev_hbm.at[pl.ds(e0 * (D // 2), BE * (D // 2))], vb, sem)
        cpd.wait()
        cps.wait()
        cpk = pltpu.async_copy(xk_hbm.at[didx], kbuf, sem)
        cpq = pltpu.async_copy(xq_hbm.at[sidx], qbuf, sem)
        cp2 = pltpu.async_copy(xv_hbm.at[sidx], v2buf, sem)
        cpg.wait()
        cpv.wait()
        cpk.wait()
        cpq.wait()
        cp2.wait()

        def edge(e, c2):
            for c in range(D // 32):
                glo = pl.ds(32 * c, L)
                ghi = pl.ds(32 * c + L, L)
                woff = pl.multiple_of(e * (D // 2) + L * c, L)
                wg_ = zb[pl.ds(woff, L)]
                wv_ = vb[pl.ds(woff, L)]
                eg_lo = plsc.bitcast(lax.shift_left(wg_, 16), jnp.float32)
                eg_hi = plsc.bitcast(jnp.bitwise_and(wg_, jnp.int32(-65536)), jnp.float32)
                ev_lo = plsc.bitcast(lax.shift_left(wv_, 16), jnp.float32)
                ev_hi = plsc.bitcast(jnp.bitwise_and(wv_, jnp.int32(-65536)), jnp.float32)
                z_lo = kbuf[e, glo] + qbuf[e, glo] + eg_lo
                z_hi = kbuf[e, ghi] + qbuf[e, ghi] + eg_hi
                s_lo = 1.0 / (1.0 + jnp.exp(-z_lo))
                s_hi = 1.0 / (1.0 + jnp.exp(-z_hi))
                mbuf[e, glo] = s_lo * (v2buf[e, glo] + ev_lo)
                mbuf[e, ghi] = s_hi * (v2buf[e, ghi] + ev_hi)
            return c2

        lax.fori_loop(0, BE, edge, 0)
        pltpu.sync_copy(mbuf, shared.at[didx], add=True)
        return carry

    lax.fori_loop(0, NIT, it_body, 0)
    plsc.subcore_barrier()

    for j in range(NCH):
        rr = pl.ds(r0 + j * RCH, RCH)
        pltpu.sync_copy(shared.at[rr], mbuf)
        pltpu.sync_copy(mbuf, out_hbm.at[core, rr])


_sc_call = pl.kernel(
    _sc_body,
    out_type=jax.ShapeDtypeStruct((NC, NPAD, D), jnp.float32),
    mesh=plsc.VectorSubcoreMesh(core_axis_name="c", subcore_axis_name="s"),
    compiler_params=pltpu.CompilerParams(needs_layout_passes=False),
    scratch_types=[
        pltpu.VMEM((BE,), jnp.int32),
        pltpu.VMEM((BE,), jnp.int32),
        pltpu.VMEM((BE, D), jnp.float32),
        pltpu.VMEM((BE, D), jnp.float32),
        pltpu.VMEM((BE, D), jnp.float32),
        pltpu.VMEM((BE * D // 2,), jnp.int32),
        pltpu.VMEM((BE * D // 2,), jnp.int32),
        pltpu.VMEM((BE, D), jnp.float32),
        pltpu.VMEM_SHARED((NPAD, D), jnp.float32),
        pltpu.SemaphoreType.DMA,
        pltpu.SemaphoreType.DMA,
    ],
)


def kernel(x, edge_index, edge_attr, Wk, bk, Wq, bq, Wv, bv, Wskip, bias):
    x_pad = jnp.pad(x, ((0, NPAD - N), (0, 0)))
    w_all = jnp.concatenate([Wk[:D], Wq[:D], Wv[:D], Wskip], axis=1)
    xk, xq, xv, skip = pl.pallas_call(
        _proj_body,
        grid=(NPAD // BN,),
        in_specs=[
            pl.BlockSpec((BN, D), lambda i: (i, 0)),
            pl.BlockSpec((D, 4 * D), lambda i: (0, 0)),
            pl.BlockSpec((1, D), lambda i: (0, 0)),
        ],
        out_specs=[pl.BlockSpec((BN, D), lambda i: (i, 0))] * 4,
        out_shape=[jax.ShapeDtypeStruct((NPAD, D), jnp.float32)] * 4,
    )(x_pad, w_all, bias.reshape(1, D))

    ea_pad = jnp.pad(edge_attr, ((0, EP - E), (0, 0)))
    src_pad = jnp.pad(edge_index[0], (0, EP - E), constant_values=N)
    dst_pad = jnp.pad(edge_index[1], (0, EP - E), constant_values=N)
    perm = jnp.asarray(_PERM)
    wg = (Wk[D:] + Wq[D:])[:, perm]
    bg = (bk + bq)[perm]
    eg, ev = pl.pallas_call(
        _edge_body,
        grid=(EP // BEB,),
        in_specs=[
            pl.BlockSpec((BEB, DE), lambda i: (i, 0)),
            pl.BlockSpec((DE, D), lambda i: (0, 0)),
            pl.BlockSpec((DE, D), lambda i: (0, 0)),
            pl.BlockSpec((1, D), lambda i: (0, 0)),
            pl.BlockSpec((1, D), lambda i: (0, 0)),
        ],
        out_specs=[pl.BlockSpec((BEB, D), lambda i: (i, 0))] * 2,
        out_shape=[jax.ShapeDtypeStruct((EP, D), jnp.bfloat16)] * 2,
    )(ea_pad, wg, Wv[D:][:, perm], bg.reshape(1, D), bv[perm].reshape(1, D))

    eg32 = lax.bitcast_convert_type(eg.reshape(EP, D // 2, 2), jnp.int32).reshape(-1)
    ev32 = lax.bitcast_convert_type(ev.reshape(EP, D // 2, 2), jnp.int32).reshape(-1)
    partial = _sc_call(xk, xq, xv, eg32, ev32, src_pad, dst_pad)

    out = pl.pallas_call(
        _comb_body,
        grid=(N // BNC,),
        in_specs=[
            pl.BlockSpec((NC, BNC, D), lambda i: (0, i, 0)),
            pl.BlockSpec((BNC, D), lambda i: (i, 0)),
        ],
        out_specs=pl.BlockSpec((BNC, D), lambda i: (i, 0)),
        out_shape=jax.ShapeDtypeStruct((N, D), jnp.float32),
    )(partial, skip)
    return out


# speedup vs baseline: 1.3704x; 1.3704x over previous
import numpy as np
import jax
import jax.numpy as jnp
from jax import lax
from jax.experimental import pallas as pl
from jax.experimental.pallas import tpu as pltpu
from jax.experimental.pallas import tpu_sc as plsc

N = 10000
E = 320000
D = 128
DE = 16

NC = 2
NS = 16
L = 16
NW = NC * NS
BE = 64
EPW = 10048
EP = EPW * NW
NIT = EPW // BE
NPAD = 10240
RPT = NPAD // NS
RCH = BE
NCH = RPT // RCH

BN = 1024
BNC = 1000
BEB = 5024

_PERM = np.empty((D,), dtype=np.int32)
for _c in range(D // 32):
    for _i in range(16):
        _PERM[32 * _c + 2 * _i] = 32 * _c + _i
        _PERM[32 * _c + 2 * _i + 1] = 32 * _c + 16 + _i


def _proj_body(x_ref, w_ref, b_ref, xk_ref, xq_ref, xv_ref, sk_ref):
    acc = jnp.dot(x_ref[...], w_ref[...], preferred_element_type=jnp.float32)
    xk_ref[...] = acc[:, 0:D]
    xq_ref[...] = acc[:, D:2 * D]
    xv_ref[...] = acc[:, 2 * D:3 * D]
    sk_ref[...] = acc[:, 3 * D:4 * D] + b_ref[...]


def _edge_body(ea_ref, wg_ref, wv_ref, bg_ref, bv_ref, eg_ref, ev_ref):
    ea = ea_ref[...]
    eg = jnp.dot(ea, wg_ref[...], preferred_element_type=jnp.float32) + bg_ref[...]
    ev = jnp.dot(ea, wv_ref[...], preferred_element_type=jnp.float32) + bv_ref[...]
    eg_ref[...] = eg.astype(jnp.bfloat16)
    ev_ref[...] = ev.astype(jnp.bfloat16)


def _comb_body(p_ref, s_ref, o_ref):
    o_ref[...] = p_ref[0] + p_ref[1] + s_ref[...]


def _sc_body(xk_hbm, xq_hbm, xv_hbm, eg_hbm, ev_hbm, src_hbm, dst_hbm, out_hbm,
             didx, sidx, kbuf, qbuf, v2buf, zb, vb, mbuf, shared,
             sem_idx, sem):
    core = lax.axis_index("c")
    sid = lax.axis_index("s")
    wid = core * NS + sid

    def zrow(r, carry):
        for c in range(D // L):
            mbuf[r, pl.ds(c * L, L)] = jnp.zeros((L,), jnp.float32)
        return carry

    lax.fori_loop(0, RCH, zrow, 0)
    r0 = sid * RPT
    for j in range(NCH):
        pltpu.sync_copy(mbuf, shared.at[pl.ds(r0 + j * RCH, RCH)])
    plsc.subcore_barrier()

    ebase = wid * EPW

    def it_body(i, carry):
        e0 = ebase + i * BE
        cpd = pltpu.async_copy(dst_hbm.at[pl.ds(e0, BE)], didx, sem_idx)
        cps = pltpu.async_copy(src_hbm.at[pl.ds(e0, BE)], sidx, sem_idx)
        cpg = pltpu.async_copy(eg_hbm.at[pl.ds(e0 * D, BE * D)], zb, sem)
        cpv = pltpu.async_copy(ev_hbm.at[pl.ds(e0 * D, BE * D)], vb, sem)
        cpd.wait()
        cps.wait()
        cpk = pltpu.async_copy(xk_hbm.at[didx], kbuf, sem)
        cpq = pltpu.async_copy(xq_hbm.at[sidx], qbuf, sem)
        cp2 = pltpu.async_copy(xv_hbm.at[sidx], v2buf, sem)
        cpg.wait()
        cpv.wait()
        cpk.wait()
        cpq.wait()
        cp2.wait()

        def edge(e, c2):
            for c in range(D // L):
                g16 = pl.ds(L * c, L)
                off = pl.multiple_of(e * D + L * c, L)
                egc = zb[pl.ds(off, L)].astype(jnp.float32)
                evc = vb[pl.ds(off, L)].astype(jnp.float32)
                z = kbuf[e, g16] + qbuf[e, g16] + egc
                s = 1.0 / (1.0 + jnp.exp(-z))
                mbuf[e, g16] = s * (v2buf[e, g16] + evc)
            return c2

        lax.fori_loop(0, BE, edge, 0)
        pltpu.sync_copy(mbuf, shared.at[didx], add=True)
        return carry

    lax.fori_loop(0, NIT, it_body, 0)
    plsc.subcore_barrier()

    for j in range(NCH):
        rr = pl.ds(r0 + j * RCH, RCH)
        pltpu.sync_copy(shared.at[rr], mbuf)
        pltpu.sync_copy(mbuf, out_hbm.at[core, rr])


_sc_call = pl.kernel(
    _sc_body,
    out_type=jax.ShapeDtypeStruct((NC, NPAD, D), jnp.float32),
    mesh=plsc.VectorSubcoreMesh(core_axis_name="c", subcore_axis_name="s"),
    scratch_types=[
        pltpu.VMEM((BE,), jnp.int32),
        pltpu.VMEM((BE,), jnp.int32),
        pltpu.VMEM((BE, D), jnp.float32),
        pltpu.VMEM((BE, D), jnp.float32),
        pltpu.VMEM((BE, D), jnp.float32),
        pltpu.VMEM((BE * D,), jnp.bfloat16),
        pltpu.VMEM((BE * D,), jnp.bfloat16),
        pltpu.VMEM((BE, D), jnp.float32),
        pltpu.VMEM_SHARED((NPAD, D), jnp.float32),
        pltpu.SemaphoreType.DMA,
        pltpu.SemaphoreType.DMA,
    ],
)


def kernel(x, edge_index, edge_attr, Wk, bk, Wq, bq, Wv, bv, Wskip, bias):
    x_pad = jnp.pad(x, ((0, NPAD - N), (0, 0)))
    w_all = jnp.concatenate([Wk[:D], Wq[:D], Wv[:D], Wskip], axis=1)
    xk, xq, xv, skip = pl.pallas_call(
        _proj_body,
        grid=(NPAD // BN,),
        in_specs=[
            pl.BlockSpec((BN, D), lambda i: (i, 0)),
            pl.BlockSpec((D, 4 * D), lambda i: (0, 0)),
            pl.BlockSpec((1, D), lambda i: (0, 0)),
        ],
        out_specs=[pl.BlockSpec((BN, D), lambda i: (i, 0))] * 4,
        out_shape=[jax.ShapeDtypeStruct((NPAD, D), jnp.float32)] * 4,
    )(x_pad, w_all, bias.reshape(1, D))

    ea_pad = jnp.pad(edge_attr, ((0, EP - E), (0, 0)))
    src_pad = jnp.pad(edge_index[0], (0, EP - E), constant_values=N)
    dst_pad = jnp.pad(edge_index[1], (0, EP - E), constant_values=N)
    wg = Wk[D:] + Wq[D:]
    bg = bk + bq
    eg, ev = pl.pallas_call(
        _edge_body,
        grid=(EP // BEB,),
        in_specs=[
            pl.BlockSpec((BEB, DE), lambda i: (i, 0)),
            pl.BlockSpec((DE, D), lambda i: (0, 0)),
            pl.BlockSpec((DE, D), lambda i: (0, 0)),
            pl.BlockSpec((1, D), lambda i: (0, 0)),
            pl.BlockSpec((1, D), lambda i: (0, 0)),
        ],
        out_specs=[pl.BlockSpec((BEB, D), lambda i: (i, 0))] * 2,
        out_shape=[jax.ShapeDtypeStruct((EP, D), jnp.bfloat16)] * 2,
    )(ea_pad, wg, Wv[D:], bg.reshape(1, D), bv.reshape(1, D))

    partial = _sc_call(xk, xq, xv, eg.reshape(-1), ev.reshape(-1), src_pad, dst_pad)

    out = pl.pallas_call(
        _comb_body,
        grid=(N // BNC,),
        in_specs=[
            pl.BlockSpec((NC, BNC, D), lambda i: (0, i, 0)),
            pl.BlockSpec((BNC, D), lambda i: (i, 0)),
        ],
        out_specs=pl.BlockSpec((BNC, D), lambda i: (i, 0)),
        out_shape=jax.ShapeDtypeStruct((N, D), jnp.float32),
    )(partial, skip)
    return out


# speedup vs baseline: 1.5446x; 1.1271x over previous
import jax
import jax.numpy as jnp
from jax import lax
from jax.experimental import pallas as pl
from jax.experimental.pallas import tpu as pltpu
from jax.experimental.pallas import tpu_sc as plsc

N = 10000
E = 320000
D = 128
DE = 16

NC = 2
NS = 16
L = 16
NW = NC * NS
BE = 32
EPW = 10048
EP = EPW * NW
NIT = EPW // BE
NPAD = 10112
RPT = NPAD // NS

BN = 1264
BNC = 1000
BEB = 5024


def _proj_body(x_ref, w_ref, b_ref, xk_ref, xqv_ref, sk_ref):
    acc = jnp.dot(x_ref[...], w_ref[...], preferred_element_type=jnp.float32)
    xk_ref[...] = acc[:, 0:D]
    xqv_ref[...] = acc[:, D:3 * D]
    sk_ref[...] = acc[:, 3 * D:4 * D] + b_ref[...]


def _edge_body(ea_ref, wgv_ref, bgv_ref, egv_ref):
    ea = ea_ref[...]
    egv_ref[...] = (
        jnp.dot(ea, wgv_ref[...], preferred_element_type=jnp.float32)
        + bgv_ref[...]
    )


def _comb_body(p_ref, s_ref, o_ref):
    o_ref[...] = p_ref[0] + p_ref[1] + s_ref[...]


def _sc_body(xk_hbm, xqv_hbm, egv_hbm, src_hbm, dst_hbm, out_hbm,
             di0, si0, di1, si1, sc0, sc1,
             kb0, kb1, qv0, qv1, eb0, eb1, mb0, mb1, shared,
             semi0, semi1, semd0, semd1, sems0, sems1):
    core = lax.axis_index("c")
    sid = lax.axis_index("s")
    wid = core * NS + sid
    ebase = wid * EPW

    sets = (
        (di0, si0, sc0, kb0, qv0, eb0, mb0, semi0, semd0, sems0),
        (di1, si1, sc1, kb1, qv1, eb1, mb1, semi1, semd1, sems1),
    )

    def zrow(r, carry):
        for c in range(D // L):
            mb0[r, pl.ds(c * L, L)] = jnp.zeros((L,), jnp.float32)
        return carry

    lax.fori_loop(0, BE, zrow, 0)
    r0 = sid * RPT
    for j in range(RPT // BE):
        pltpu.sync_copy(mb0, shared.at[pl.ds(r0 + j * BE, BE)])
    rem = RPT - (RPT // BE) * BE
    if rem:
        pltpu.sync_copy(mb0.at[pl.ds(0, rem)],
                        shared.at[pl.ds(r0 + (RPT // BE) * BE, rem)])
    plsc.subcore_barrier()

    def batch_off(j):
        return ebase + lax.rem(j, NIT) * BE

    def issue_idx(j, s):
        e0 = batch_off(j)
        pltpu.async_copy(dst_hbm.at[pl.ds(e0, BE)], s[0], s[7])
        pltpu.async_copy(src_hbm.at[pl.ds(e0, BE)], s[1], s[7])

    def wait_idx(s):
        pltpu.make_async_copy(dst_hbm.at[pl.ds(0, BE)], s[0], s[7]).wait()
        pltpu.make_async_copy(src_hbm.at[pl.ds(0, BE)], s[1], s[7]).wait()

    def issue_data(j, s):
        e0 = batch_off(j)
        pltpu.async_copy(egv_hbm.at[pl.ds(e0, BE)], s[5], s[8])
        pltpu.async_copy(xk_hbm.at[s[0]], s[3], s[8])
        pltpu.async_copy(xqv_hbm.at[s[1]], s[4], s[8])

    def wait_data(s):
        pltpu.make_async_copy(egv_hbm.at[pl.ds(0, BE)], s[5], s[8]).wait()
        pltpu.make_async_copy(xk_hbm.at[s[0]], s[3], s[8]).wait()
        pltpu.make_async_copy(xqv_hbm.at[s[1]], s[4], s[8]).wait()

    def wait_scatter(s):
        pltpu.make_async_copy(s[6], shared.at[s[2]], s[9]).wait()

    def compute_and_scatter(s):
        didx, sidx, scidx, kb, qv, eb, mb = s[0], s[1], s[2], s[3], s[4], s[5], s[6]

        def edge(e, carry):
            for c in range(D // L):
                sl = pl.ds(L * c, L)
                sv = pl.ds(D + L * c, L)
                z = kb[e, sl] + qv[e, sl] + eb[e, sl]
                g = 1.0 / (1.0 + jnp.exp(-z))
                mb[e, sl] = g * (qv[e, sv] + eb[e, sv])
            return carry

        lax.fori_loop(0, BE, edge, 0)
        for c in range(BE // L):
            scidx[pl.ds(c * L, L)] = didx[pl.ds(c * L, L)]
        pltpu.async_copy(mb, shared.at[scidx], s[9], add=True)

    def body(j, cur, nxt, first):
        wait_idx(nxt)
        issue_data(j + 1, nxt)
        wait_data(cur)
        if not first:
            wait_scatter(cur)
        compute_and_scatter(cur)
        issue_idx(j + 2, cur)

    issue_idx(0, sets[0])
    issue_idx(1, sets[1])
    wait_idx(sets[0])
    issue_data(0, sets[0])
    body(0, sets[0], sets[1], True)
    body(1, sets[1], sets[0], True)

    def loop_body(jj, carry):
        j = 2 * jj + 2
        body(j, sets[0], sets[1], False)
        body(j + 1, sets[1], sets[0], False)
        return carry

    lax.fori_loop(0, (NIT - 2) // 2, loop_body, 0)

    wait_data(sets[NIT % 2])
    wait_idx(sets[(NIT + 1) % 2])
    wait_scatter(sets[0])
    wait_scatter(sets[1])
    plsc.subcore_barrier()

    for j in range(RPT // BE):
        rr = pl.ds(r0 + j * BE, BE)
        pltpu.sync_copy(shared.at[rr], mb0)
        pltpu.sync_copy(mb0, out_hbm.at[core, rr])
    if rem:
        rr = pl.ds(r0 + (RPT // BE) * BE, rem)
        pltpu.sync_copy(shared.at[rr], mb0.at[pl.ds(0, rem)])
        pltpu.sync_copy(mb0.at[pl.ds(0, rem)], out_hbm.at[core, rr])


_sc_call = pl.kernel(
    _sc_body,
    out_type=jax.ShapeDtypeStruct((NC, NPAD, D), jnp.float32),
    mesh=plsc.VectorSubcoreMesh(core_axis_name="c", subcore_axis_name="s"),
    scratch_types=[
        pltpu.VMEM((BE,), jnp.int32),
        pltpu.VMEM((BE,), jnp.int32),
        pltpu.VMEM((BE,), jnp.int32),
        pltpu.VMEM((BE,), jnp.int32),
        pltpu.VMEM((BE,), jnp.int32),
        pltpu.VMEM((BE,), jnp.int32),
        pltpu.VMEM((BE, D), jnp.float32),
        pltpu.VMEM((BE, D), jnp.float32),
        pltpu.VMEM((BE, 2 * D), jnp.float32),
        pltpu.VMEM((BE, 2 * D), jnp.float32),
        pltpu.VMEM((BE, 2 * D), jnp.float32),
        pltpu.VMEM((BE, 2 * D), jnp.float32),
        pltpu.VMEM((BE, D), jnp.float32),
        pltpu.VMEM((BE, D), jnp.float32),
        pltpu.VMEM_SHARED((NPAD, D), jnp.float32),
        pltpu.SemaphoreType.DMA,
        pltpu.SemaphoreType.DMA,
        pltpu.SemaphoreType.DMA,
        pltpu.SemaphoreType.DMA,
        pltpu.SemaphoreType.DMA,
        pltpu.SemaphoreType.DMA,
    ],
)


def kernel(x, edge_index, edge_attr, Wk, bk, Wq, bq, Wv, bv, Wskip, bias):
    x_pad = jnp.pad(x, ((0, NPAD - N), (0, 0)))
    w_all = jnp.concatenate([Wk[:D], Wq[:D], Wv[:D], Wskip], axis=1)
    xk, xqv, skip = pl.pallas_call(
        _proj_body,
        grid=(NPAD // BN,),
        in_specs=[
            pl.BlockSpec((BN, D), lambda i: (i, 0)),
            pl.BlockSpec((D, 4 * D), lambda i: (0, 0)),
            pl.BlockSpec((1, D), lambda i: (0, 0)),
        ],
        out_specs=[
            pl.BlockSpec((BN, D), lambda i: (i, 0)),
            pl.BlockSpec((BN, 2 * D), lambda i: (i, 0)),
            pl.BlockSpec((BN, D), lambda i: (i, 0)),
        ],
        out_shape=[
            jax.ShapeDtypeStruct((NPAD, D), jnp.float32),
            jax.ShapeDtypeStruct((NPAD, 2 * D), jnp.float32),
            jax.ShapeDtypeStruct((NPAD, D), jnp.float32),
        ],
    )(x_pad, w_all, bias.reshape(1, D))

    ea_pad = jnp.pad(edge_attr, ((0, EP - E), (0, 0)))
    src_pad = jnp.pad(edge_index[0], (0, EP - E), constant_values=N)
    dst_pad = jnp.pad(edge_index[1], (0, EP - E), constant_values=N)
    wgv = jnp.concatenate([Wk[D:] + Wq[D:], Wv[D:]], axis=1)
    bgv = jnp.concatenate([bk + bq, bv])
    egv = pl.pallas_call(
        _edge_body,
        grid=(EP // BEB,),
        in_specs=[
            pl.BlockSpec((BEB, DE), lambda i: (i, 0)),
            pl.BlockSpec((DE, 2 * D), lambda i: (0, 0)),
            pl.BlockSpec((1, 2 * D), lambda i: (0, 0)),
        ],
        out_specs=pl.BlockSpec((BEB, 2 * D), lambda i: (i, 0)),
        out_shape=jax.ShapeDtypeStruct((EP, 2 * D), jnp.float32),
    )(ea_pad, wgv, bgv.reshape(1, 2 * D))

    partial = _sc_call(xk, xqv, egv, src_pad, dst_pad)

    out = pl.pallas_call(
        _comb_body,
        grid=(N // BNC,),
        in_specs=[
            pl.BlockSpec((NC, BNC, D), lambda i: (0, i, 0)),
            pl.BlockSpec((BNC, D), lambda i: (i, 0)),
        ],
        out_specs=pl.BlockSpec((BNC, D), lambda i: (i, 0)),
        out_shape=jax.ShapeDtypeStruct((N, D), jnp.float32),
    )(partial, skip)
    return out
